# baseline (speedup 1.0000x reference)
import functools

import numpy as np
import jax
import jax.numpy as jnp
from jax import lax
from jax.experimental import pallas as pl
from jax.experimental.pallas import tpu as pltpu
from jax.experimental.pallas import tpu_sc as plsc

_ARITY = 4
_GROUPS = 256
_OUT_DIM = 4
_LANES = 16
_NC = 2
_NS = 16
_NW = _NC * _NS


def _mask_tables():
    m1 = np.zeros((_ARITY, 8), dtype=np.int32)
    for p in range(_ARITY):
        others = [q for q in range(_ARITY) if q != p]
        for j in range(8):
            m = 1 << p
            for t in range(3):
                if (j >> t) & 1:
                    m += 1 << others[t]
            m1[p, j] = m
    m0 = m1 - (1 << np.arange(_ARITY, dtype=np.int32))[:, None]
    return m1, m0


_M1, _M0 = _mask_tables()


def _build_dtab(params):
    keep = jnp.asarray((_M0 > 0), params.dtype)[None, :, :, None]
    d = params[:, _M1, :] - keep * params[:, _M0, :]
    return jnp.transpose(d, (1, 2, 3, 0)).reshape(-1)


@functools.lru_cache(maxsize=None)
def _make_sc_kernel(batch: int, width: int):
    rows_per_w = batch // _NW
    R = 16
    n_chunks = rows_per_w // R
    assert rows_per_w % R == 0 and n_chunks % 2 == 0
    vecs = width // (_ARITY * _LANES)
    assert vecs == 16

    mesh = plsc.VectorSubcoreMesh(core_axis_name="c", subcore_axis_name="s")

    @functools.partial(
        pl.kernel,
        mesh=mesh,
        compiler_params=pltpu.CompilerParams(
            use_tc_tiling_on_sc=True, needs_layout_passes=False),
        out_type=jax.ShapeDtypeStruct((batch, width), jnp.float32),
        scratch_types=[
            pltpu.VMEM((_GROUPS * 128,), jnp.float32),
            pltpu.VMEM((2, R, width), jnp.float32),
            pltpu.VMEM((2, R, width), jnp.float32),
            pltpu.SemaphoreType.DMA,
            pltpu.SemaphoreType.DMA,
            pltpu.SemaphoreType.DMA,
            pltpu.SemaphoreType.DMA,
        ],
    )
    def hoa(x_hbm, dtab_hbm, out_hbm, dtab_v, x_v, o_v, si0, si1, so0, so1):
        wid = lax.axis_index("s") * _NC + lax.axis_index("c")
        row0 = wid * rows_per_w
        in_sems = (si0, si1)
        out_sems = (so0, so1)

        pltpu.sync_copy(dtab_hbm, dtab_v)

        iota = lax.iota(jnp.int32, _LANES)
        i4 = iota * 4
        ones = jnp.ones((_LANES,), jnp.int32)
        zeros = jnp.zeros((_LANES,), jnp.int32)

        def in_copy(c, buf):
            return pltpu.make_async_copy(
                x_hbm.at[pl.ds(row0 + c * R, R)], x_v.at[buf], in_sems[buf])

        def out_copy(c, buf):
            return pltpu.make_async_copy(
                o_v.at[buf], out_hbm.at[pl.ds(row0 + c * R, R)], out_sems[buf])

        _dt_len = (96 + 7 * 4) * _GROUPS + _GROUPS
        dtab_lv = [dtab_v.at[pl.ds(l * _GROUPS, _dt_len)]
                   for l in range(_OUT_DIM)]

        def compute_chunk(buf):
            xb = x_v.at[buf]
            ob = o_v.at[buf]

            @plsc.parallel_loop(0, R * vecs, unroll=3)
            def _(i):
                r = i >> 4
                v = i & 15
                ir = zeros + r
                if True:
                    cb = i4 + v * (_ARITY * _LANES)
                    a0 = plsc.load_gather(xb, [ir, cb])
                    a1 = plsc.load_gather(xb, [ir, cb + 1])
                    a2 = plsc.load_gather(xb, [ir, cb + 2])
                    a3 = plsc.load_gather(xb, [ir, cb + 3])
                    s01 = jnp.where(a1 >= a0, ones, zeros)
                    s02 = jnp.where(a2 >= a0, ones, zeros)
                    s03 = jnp.where(a3 >= a0, ones, zeros)
                    s12 = jnp.where(a2 >= a1, ones, zeros)
                    s13 = jnp.where(a3 >= a1, ones, zeros)
                    s23 = jnp.where(a3 >= a2, ones, zeros)
                    t12 = s12 * 2
                    t13 = s13 * 2
                    t23 = s23 * 4
                    j0 = s01 + s02 * 2 + s03 * 4
                    j1 = (ones - s01) + t12 + s13 * 4
                    j2 = (3 - s02) - t12 + t23
                    j3 = (7 - s03) - t13 - t23
                    gv = iota + v * _LANES
                    t0 = j0 * (4 * _GROUPS) + gv
                    t1 = j1 * (4 * _GROUPS) + (gv + 32 * _GROUPS)
                    t2 = j2 * (4 * _GROUPS) + (gv + 64 * _GROUPS)
                    t3 = j3 * (4 * _GROUPS) + (gv + 96 * _GROUPS)
                    for l in range(_OUT_DIM):
                        dt = dtab_lv[l]
                        d0 = plsc.load_gather(dt, [t0])
                        d1 = plsc.load_gather(dt, [t1])
                        d2 = plsc.load_gather(dt, [t2])
                        d3 = plsc.load_gather(dt, [t3])
                        acc = (a0 * d0 + a1 * d1) + (a2 * d2 + a3 * d3)
                        plsc.store_scatter(ob, [ir, cb + l], acc)

        in_copy(0, 0).start()

        def pair_body(pair, carry):
            for b in (0, 1):
                c = pair * 2 + b
                in_copy(c, b).wait()

                @pl.when(c + 1 < n_chunks)
                def _():
                    in_copy(c + 1, 1 - b).start()

                @pl.when(c >= 2)
                def _():
                    out_copy(c - 2, b).wait()

                compute_chunk(b)
                out_copy(c, b).start()
            return carry

        lax.fori_loop(0, n_chunks // 2, pair_body, 0)
        out_copy(n_chunks - 2, 0).wait()
        out_copy(n_chunks - 1, 1).wait()

    return hoa


def kernel(X, params):
    batch, width = X.shape
    dtab = _build_dtab(params.astype(jnp.float32))
    fn = _make_sc_kernel(batch, width)
    return fn(X.astype(jnp.float32), dtab)


# speedup vs baseline: 1.0203x; 1.0203x over previous
import functools

import numpy as np
import jax
import jax.numpy as jnp
from jax import lax
from jax.experimental import pallas as pl
from jax.experimental.pallas import tpu as pltpu
from jax.experimental.pallas import tpu_sc as plsc

_ARITY = 4
_GROUPS = 256
_OUT_DIM = 4
_LANES = 16
_NC = 2
_NS = 16
_NW = _NC * _NS


def _mask_tables():
    m1 = np.zeros((_ARITY, 8), dtype=np.int32)
    for p in range(_ARITY):
        others = [q for q in range(_ARITY) if q != p]
        for j in range(8):
            m = 1 << p
            for t in range(3):
                if (j >> t) & 1:
                    m += 1 << others[t]
            m1[p, j] = m
    m0 = m1 - (1 << np.arange(_ARITY, dtype=np.int32))[:, None]
    return m1, m0


_M1, _M0 = _mask_tables()


def _build_dtab(params):
    keep = jnp.asarray((_M0 > 0), params.dtype)[None, :, :, None]
    d = params[:, _M1, :] - keep * params[:, _M0, :]
    return jnp.transpose(d, (1, 2, 3, 0)).reshape(-1)


@functools.lru_cache(maxsize=None)
def _make_sc_kernel(batch: int, width: int):
    rows_per_w = batch // _NW
    R = 16
    n_chunks = rows_per_w // R
    assert rows_per_w % R == 0 and n_chunks % 2 == 0
    vecs = width // (_ARITY * _LANES)
    assert vecs == 16

    mesh = plsc.VectorSubcoreMesh(core_axis_name="c", subcore_axis_name="s")

    @functools.partial(
        pl.kernel,
        mesh=mesh,
        compiler_params=pltpu.CompilerParams(
            use_tc_tiling_on_sc=True, needs_layout_passes=False),
        out_type=jax.ShapeDtypeStruct((batch, width), jnp.float32),
        scratch_types=[
            pltpu.VMEM((_GROUPS * 128,), jnp.float32),
            pltpu.VMEM((2, R, width), jnp.float32),
            pltpu.VMEM((2, R, width), jnp.float32),
            pltpu.SemaphoreType.DMA,
            pltpu.SemaphoreType.DMA,
            pltpu.SemaphoreType.DMA,
            pltpu.SemaphoreType.DMA,
        ],
    )
    def hoa(x_hbm, dtab_hbm, out_hbm, dtab_v, x_v, o_v, si0, si1, so0, so1):
        wid = lax.axis_index("s") * _NC + lax.axis_index("c")
        row0 = wid * rows_per_w
        in_sems = (si0, si1)
        out_sems = (so0, so1)

        pltpu.sync_copy(dtab_hbm, dtab_v)

        iota = lax.iota(jnp.int32, _LANES)
        i4 = iota * 4
        ones = jnp.ones((_LANES,), jnp.int32)
        zeros = jnp.zeros((_LANES,), jnp.int32)

        def in_copy(c, buf):
            return pltpu.make_async_copy(
                x_hbm.at[pl.ds(row0 + c * R, R)], x_v.at[buf], in_sems[buf])

        def out_copy(c, buf):
            return pltpu.make_async_copy(
                o_v.at[buf], out_hbm.at[pl.ds(row0 + c * R, R)], out_sems[buf])

        _dt_len = (96 + 7 * 4) * _GROUPS + _GROUPS
        dtab_lv = [dtab_v.at[pl.ds(l * _GROUPS, _dt_len)]
                   for l in range(_OUT_DIM)]

        def compute_chunk(buf):
            xb = x_v.at[buf]
            ob = o_v.at[buf]

            @plsc.parallel_loop(0, R * vecs, unroll=3)
            def _(i):
                r = i >> 4
                v = i & 15
                ir = zeros + r
                if True:
                    cb = i4 + v * (_ARITY * _LANES)
                    a0 = plsc.load_gather(xb, [ir, cb])
                    a1 = plsc.load_gather(xb, [ir, cb + 1])
                    a2 = plsc.load_gather(xb, [ir, cb + 2])
                    a3 = plsc.load_gather(xb, [ir, cb + 3])
                    s01 = jnp.where(a1 >= a0, ones, zeros)
                    s02 = jnp.where(a2 >= a0, ones, zeros)
                    s03 = jnp.where(a3 >= a0, ones, zeros)
                    s12 = jnp.where(a2 >= a1, ones, zeros)
                    s13 = jnp.where(a3 >= a1, ones, zeros)
                    s23 = jnp.where(a3 >= a2, ones, zeros)
                    t12 = s12 * 2
                    t13 = s13 * 2
                    t23 = s23 * 4
                    j0 = s01 + s02 * 2 + s03 * 4
                    j1 = (ones - s01) + t12 + s13 * 4
                    j2 = (3 - s02) - t12 + t23
                    j3 = (7 - s03) - t13 - t23
                    gv = iota + v * _LANES
                    t0 = j0 * (4 * _GROUPS) + gv
                    t1 = j1 * (4 * _GROUPS) + (gv + 32 * _GROUPS)
                    t2 = j2 * (4 * _GROUPS) + (gv + 64 * _GROUPS)
                    t3 = j3 * (4 * _GROUPS) + (gv + 96 * _GROUPS)
                    for l in range(_OUT_DIM):
                        dt = dtab_lv[l]
                        d0 = plsc.load_gather(dt, [t0])
                        d1 = plsc.load_gather(dt, [t1])
                        d2 = plsc.load_gather(dt, [t2])
                        d3 = plsc.load_gather(dt, [t3])
                        acc = a0 * d0 + a1 * d1 + a2 * d2 + a3 * d3
                        plsc.store_scatter(ob, [ir, cb + l], acc)

        in_copy(0, 0).start()

        def pair_body(pair, carry):
            for b in (0, 1):
                c = pair * 2 + b
                in_copy(c, b).wait()

                @pl.when(c + 1 < n_chunks)
                def _():
                    in_copy(c + 1, 1 - b).start()

                @pl.when(c >= 2)
                def _():
                    out_copy(c - 2, b).wait()

                compute_chunk(b)
                out_copy(c, b).start()
            return carry

        lax.fori_loop(0, n_chunks // 2, pair_body, 0)
        out_copy(n_chunks - 2, 0).wait()
        out_copy(n_chunks - 1, 1).wait()

    return hoa


def kernel(X, params):
    batch, width = X.shape
    dtab = _build_dtab(params.astype(jnp.float32))
    fn = _make_sc_kernel(batch, width)
    return fn(X.astype(jnp.float32), dtab)
